# baseline (device time: 72378 ns/iter reference)
import jax
import jax.numpy as jnp
from jax import lax
from jax.experimental import pallas as pl
from jax.experimental.pallas import tpu as pltpu

N_DEV = 8
B, SQ, SKV, D = 4, 256, 1024, 1024
HQ_PER = 8
HKV_PER = 2
DH = 128
SCALE = 0.08838834764831843

ROWS = B * SQ

import os as _os
_SKIP_COMM = _os.environ.get("KERNEL_SKIP_COMM") == "1"

_PARTS = (
    (0, 256, "xyz"),
    (256, 256, "yzx"),
    (512, 256, "zxy"),
    (768, 256, "xyz"),
)
_COMM_OFF = []
_off = 0
for _, _n, _ in _PARTS:
    offs = []
    for _s in range(3):
        offs.append(_off)
        _off += _n >> (_s + 1)
    _COMM_OFF.append(tuple(offs))
_COMM_ROWS = _off


def _fused_body(x_ref, wq_ref, wo_ref, kext_ref, vext_ref, out_ref,
                comm_ref, kbuf, vbuf, kv_sems, send_sems, recv_sems):
    my = lax.axis_index("i")
    bit = {
        "x": (my ^ (my >> 1)) & 1,
        "y": (my >> 1) & 1,
        "z": (my >> 2) & 1,
    }
    partner = {"x": my ^ 1, "y": my ^ 3, "z": my ^ 4}

    barrier_sem = pltpu.get_barrier_semaphore()
    for d in ("x", "y", "z"):
        pl.semaphore_signal(
            barrier_sem, inc=1,
            device_id=(partner[d],), device_id_type=pl.DeviceIdType.MESH,
        )
    pl.semaphore_wait(barrier_sem, 3)

    copies = []
    for b in range(B):
        for g in range(HKV_PER):
            h = HKV_PER * my + g
            kc = pltpu.make_async_copy(
                kext_ref.at[b, :, h, :], kbuf.at[b, g],
                kv_sems.at[2 * (b * HKV_PER + g)],
            )
            vc = pltpu.make_async_copy(
                vext_ref.at[b, :, h, :], vbuf.at[b, g],
                kv_sems.at[2 * (b * HKV_PER + g) + 1],
            )
            kc.start()
            vc.start()
            copies.extend((kc, vc))

    Q = jnp.dot(
        x_ref[:, :].astype(jnp.bfloat16), wq_ref[:, :].astype(jnp.bfloat16),
        preferred_element_type=jnp.float32,
    ) * SCALE

    for c in copies:
        c.wait()

    wo_b = wo_ref[:, :].astype(jnp.bfloat16)

    base = [my * 0 + rb for rb, _, _ in _PARTS]
    rs_state = [None] * len(_PARTS)
    ag_state = [None] * len(_PARTS)

    def rs_start(pi, s):
        _, nrows, order = _PARTS[pi]
        d = order[s]
        half = nrows >> (s + 1)
        keep_off = base[pi] + bit[d] * half
        send_off = base[pi] + (1 - bit[d]) * half
        rdma = pltpu.make_async_remote_copy(
            src_ref=out_ref.at[pl.ds(send_off, half), :],
            dst_ref=comm_ref.at[pl.ds(_COMM_OFF[pi][s], half), :],
            send_sem=send_sems.at[pi * 6 + s],
            recv_sem=recv_sems.at[pi * 6 + s],
            device_id=(partner[d],),
            device_id_type=pl.DeviceIdType.MESH,
        )
        rdma.start()
        rs_state[pi] = (rdma, keep_off, half, _COMM_OFF[pi][s])

    def rs_finish(pi):
        rdma, keep_off, half, coff = rs_state[pi]
        rdma.wait()
        out_ref[pl.ds(keep_off, half), :] += comm_ref[pl.ds(coff, half), :]
        base[pi] = keep_off

    def ag_start(pi, s):
        _, nrows, order = _PARTS[pi]
        d = order[2 - s]
        blk = nrows >> (3 - s)
        rdma = pltpu.make_async_remote_copy(
            src_ref=out_ref.at[pl.ds(base[pi], blk), :],
            dst_ref=out_ref.at[pl.ds(base[pi], blk), :],
            send_sem=send_sems.at[pi * 6 + 3 + s],
            recv_sem=recv_sems.at[pi * 6 + 3 + s],
            device_id=(partner[d],),
            device_id_type=pl.DeviceIdType.MESH,
        )
        rdma.start()
        ag_state[pi] = (rdma, blk, bit[d])

    def ag_finish(pi):
        rdma, blk, b = ag_state[pi]
        rdma.wait()
        base[pi] = base[pi] - b * blk

    progress = [-1] * len(_PARTS)

    def start(pi, k):
        if k < 3:
            rs_start(pi, k)
        else:
            ag_start(pi, k - 3)
        progress[pi] = k

    def pump(pi):
        k = progress[pi]
        if k < 3:
            rs_finish(pi)
        else:
            ag_finish(pi)
        if k < 5:
            start(pi, k + 1)

    for b in range(B):
        os = []
        for g in range(HKV_PER):
            kbg = kbuf[b, g, :, :].astype(jnp.bfloat16)
            vbg = vbuf[b, g, :, :].astype(jnp.bfloat16)
            for r in range(HQ_PER // HKV_PER):
                t = g * (HQ_PER // HKV_PER) + r
                qh = Q[b * SQ:(b + 1) * SQ, t * DH:(t + 1) * DH]
                s = lax.dot_general(
                    qh.astype(jnp.bfloat16), kbg,
                    (((1,), (1,)), ((), ())),
                    preferred_element_type=jnp.float32,
                )
                e = jnp.exp(s)
                l = jnp.sum(e, axis=1, keepdims=True)
                o = jnp.dot(e.astype(jnp.bfloat16), vbg,
                            preferred_element_type=jnp.float32)
                os.append((o / l).astype(jnp.bfloat16))
        attn_b = jnp.concatenate(os, axis=1)
        out_ref[b * SQ:(b + 1) * SQ, :] = jnp.dot(
            attn_b, wo_b, preferred_element_type=jnp.float32)
        if not _SKIP_COMM:
            start(b, 0)
            if b == 2:
                pump(0)
            elif b == 3:
                pump(0)
                pump(1)

    if not _SKIP_COMM:
        remaining = [5 - progress[pi] + 1 for pi in range(len(_PARTS))]
        while any(r > 0 for r in remaining):
            for pi in range(len(_PARTS)):
                if remaining[pi] > 0:
                    pump(pi)
                    remaining[pi] -= 1


def kernel(x, Wq, Wo, K_ext, V_ext):
    out = pl.pallas_call(
        _fused_body,
        out_shape=jax.ShapeDtypeStruct((ROWS, D), jnp.float32),
        in_specs=[
            pl.BlockSpec(memory_space=pltpu.VMEM),
            pl.BlockSpec(memory_space=pltpu.VMEM),
            pl.BlockSpec(memory_space=pltpu.VMEM),
            pl.BlockSpec(memory_space=pl.ANY),
            pl.BlockSpec(memory_space=pl.ANY),
        ],
        out_specs=pl.BlockSpec(memory_space=pltpu.VMEM),
        scratch_shapes=[
            pltpu.VMEM((_COMM_ROWS, D), jnp.float32),
            pltpu.VMEM((B, HKV_PER, SKV, DH), jnp.float32),
            pltpu.VMEM((B, HKV_PER, SKV, DH), jnp.float32),
            pltpu.SemaphoreType.DMA((2 * B * HKV_PER,)),
            pltpu.SemaphoreType.DMA((6 * len(_PARTS),)),
            pltpu.SemaphoreType.DMA((6 * len(_PARTS),)),
        ],
        compiler_params=pltpu.CompilerParams(collective_id=0),
    )(x.reshape(ROWS, D), Wq, Wo, K_ext, V_ext)
    return out.reshape(B, SQ, D)


# device time: 30067 ns/iter; 2.4072x vs baseline; 2.4072x over previous
import jax
import jax.numpy as jnp
from jax import lax
from jax.experimental import pallas as pl
from jax.experimental.pallas import tpu as pltpu

N_DEV = 8
B, SQ, SKV, D = 4, 256, 1024, 1024
HQ_PER = 8
HKV_PER = 2
DH = 128
SCALE = 0.08838834764831843

ROWS = B * SQ

import os as _os
_SKIP_COMM = _os.environ.get("KERNEL_SKIP_COMM") == "1"

_PARTS = (
    (0, 256, "xyz"),
    (256, 256, "yzx"),
    (512, 256, "zxy"),
    (768, 256, "xyz"),
)
_COMM_OFF = []
_off = 0
for _, _n, _ in _PARTS:
    offs = []
    for _s in range(3):
        offs.append(_off)
        _off += _n >> (_s + 1)
    _COMM_OFF.append(tuple(offs))
_COMM_ROWS = _off


def _fused_body(x_ref, wq_ref, wo_ref, kext_ref, vext_ref, out_ref,
                comm_ref, kbuf, vbuf, kv_sems, send_sems, recv_sems):
    my = lax.axis_index("i")
    bit = {
        "x": (my ^ (my >> 1)) & 1,
        "y": (my >> 1) & 1,
        "z": (my >> 2) & 1,
    }
    partner = {"x": my ^ 1, "y": my ^ 3, "z": my ^ 4}

    barrier_sem = pltpu.get_barrier_semaphore()
    for d in ("x", "y", "z"):
        pl.semaphore_signal(
            barrier_sem, inc=1,
            device_id=(partner[d],), device_id_type=pl.DeviceIdType.MESH,
        )
    pl.semaphore_wait(barrier_sem, 3)

    copies = []
    for b in range(B):
        for g in range(HKV_PER):
            h = HKV_PER * my + g
            kc = pltpu.make_async_copy(
                kext_ref.at[b, :, h, :], kbuf.at[b, g],
                kv_sems.at[2 * (b * HKV_PER + g)],
            )
            vc = pltpu.make_async_copy(
                vext_ref.at[b, :, h, :], vbuf.at[b, g],
                kv_sems.at[2 * (b * HKV_PER + g) + 1],
            )
            kc.start()
            vc.start()
            copies.extend((kc, vc))

    Q = jnp.dot(
        x_ref[:, :].astype(jnp.bfloat16), wq_ref[:, :].astype(jnp.bfloat16),
        preferred_element_type=jnp.float32,
    ) * SCALE

    for c in copies:
        c.wait()

    wo_b = wo_ref[:, :].astype(jnp.bfloat16)

    base = [my * 0 + rb for rb, _, _ in _PARTS]
    rs_state = [None] * len(_PARTS)
    ag_state = [None] * len(_PARTS)

    def rs_start(pi, s):
        _, nrows, order = _PARTS[pi]
        d = order[s]
        half = nrows >> (s + 1)
        keep_off = base[pi] + bit[d] * half
        send_off = base[pi] + (1 - bit[d]) * half
        rdma = pltpu.make_async_remote_copy(
            src_ref=out_ref.at[pl.ds(send_off, half), :],
            dst_ref=comm_ref.at[pl.ds(_COMM_OFF[pi][s], half), :],
            send_sem=send_sems.at[pi * 6 + s],
            recv_sem=recv_sems.at[pi * 6 + s],
            device_id=(partner[d],),
            device_id_type=pl.DeviceIdType.MESH,
        )
        rdma.start()
        rs_state[pi] = (rdma, keep_off, half, _COMM_OFF[pi][s])

    def rs_finish(pi):
        rdma, keep_off, half, coff = rs_state[pi]
        rdma.wait()
        out_ref[pl.ds(keep_off, half), :] += comm_ref[pl.ds(coff, half), :]
        base[pi] = keep_off

    def ag_start(pi, s):
        _, nrows, order = _PARTS[pi]
        d = order[2 - s]
        blk = nrows >> (3 - s)
        rdma = pltpu.make_async_remote_copy(
            src_ref=out_ref.at[pl.ds(base[pi], blk), :],
            dst_ref=out_ref.at[pl.ds(base[pi], blk), :],
            send_sem=send_sems.at[pi * 6 + 3 + s],
            recv_sem=recv_sems.at[pi * 6 + 3 + s],
            device_id=(partner[d],),
            device_id_type=pl.DeviceIdType.MESH,
        )
        rdma.start()
        ag_state[pi] = (rdma, blk, bit[d])

    def ag_finish(pi):
        rdma, blk, b = ag_state[pi]
        rdma.wait()
        base[pi] = base[pi] - b * blk

    progress = [-1] * len(_PARTS)

    def start(pi, k):
        if k < 3:
            rs_start(pi, k)
        else:
            ag_start(pi, k - 3)
        progress[pi] = k

    def pump(pi):
        k = progress[pi]
        if k < 3:
            rs_finish(pi)
        else:
            ag_finish(pi)
        if k < 5:
            start(pi, k + 1)

    for b in range(B):
        os = []
        for g in range(HKV_PER):
            kbg = kbuf[b, g, :, :].astype(jnp.bfloat16)
            vbg = vbuf[b, g, :, :].astype(jnp.bfloat16)
            for r in range(HQ_PER // HKV_PER):
                t = g * (HQ_PER // HKV_PER) + r
                qh = Q[b * SQ:(b + 1) * SQ, t * DH:(t + 1) * DH]
                s = lax.dot_general(
                    qh.astype(jnp.bfloat16), kbg,
                    (((1,), (1,)), ((), ())),
                    preferred_element_type=jnp.float32,
                )
                e = jnp.exp(s)
                l = jnp.sum(e, axis=1, keepdims=True)
                o = jnp.dot(e.astype(jnp.bfloat16), vbg,
                            preferred_element_type=jnp.float32)
                os.append((o / l).astype(jnp.bfloat16))
        attn_b = jnp.concatenate(os, axis=1)
        out_ref[b * SQ:(b + 1) * SQ, :] = jnp.dot(
            attn_b, wo_b, preferred_element_type=jnp.float32)
        if not _SKIP_COMM:
            start(b, 0)

    if not _SKIP_COMM:
        for _ in range(6):
            for pi in range(len(_PARTS)):
                pump(pi)


def kernel(x, Wq, Wo, K_ext, V_ext):
    out = pl.pallas_call(
        _fused_body,
        out_shape=jax.ShapeDtypeStruct((ROWS, D), jnp.float32),
        in_specs=[
            pl.BlockSpec(memory_space=pltpu.VMEM),
            pl.BlockSpec(memory_space=pltpu.VMEM),
            pl.BlockSpec(memory_space=pltpu.VMEM),
            pl.BlockSpec(memory_space=pl.ANY),
            pl.BlockSpec(memory_space=pl.ANY),
        ],
        out_specs=pl.BlockSpec(memory_space=pltpu.VMEM),
        scratch_shapes=[
            pltpu.VMEM((_COMM_ROWS, D), jnp.float32),
            pltpu.VMEM((B, HKV_PER, SKV, DH), jnp.float32),
            pltpu.VMEM((B, HKV_PER, SKV, DH), jnp.float32),
            pltpu.SemaphoreType.DMA((2 * B * HKV_PER,)),
            pltpu.SemaphoreType.DMA((6 * len(_PARTS),)),
            pltpu.SemaphoreType.DMA((6 * len(_PARTS),)),
        ],
        compiler_params=pltpu.CompilerParams(collective_id=0),
    )(x.reshape(ROWS, D), Wq, Wo, K_ext, V_ext)
    return out.reshape(B, SQ, D)
